# baseline (speedup 1.0000x reference)
import functools

import jax
import jax.numpy as jnp
from jax import lax
from jax.experimental import pallas as pl
from jax.experimental.pallas import tpu as pltpu
from jax.experimental.pallas import tpu_sc as plsc

N = 10000
E = 320000
D = 128

_info = plsc.get_sparse_core_info()
NC = _info.num_cores
NS = _info.num_subcores
NW = NC * NS

Q = 2
ESEG = E // Q
EPW = ESEG // NW
CHUNK = 128
NFULL = EPW // CHUNK
TAIL = EPW - NFULL * CHUNK

N_PAD = 10240
ROWS_PER_TILE = N_PAD // NS

_MESH = plsc.VectorSubcoreMesh(core_axis_name="c", subcore_axis_name="s")


@functools.partial(
    pl.kernel,
    out_type=(
        jax.ShapeDtypeStruct((ESEG, D), jnp.float32),
        jax.ShapeDtypeStruct((ESEG, D), jnp.float32),
    ),
    mesh=_MESH,
    scratch_types=[
        pltpu.VMEM((EPW,), jnp.int32),
        pltpu.VMEM((EPW,), jnp.int32),
        pltpu.VMEM((CHUNK, D), jnp.float32),
        pltpu.VMEM((CHUNK, D), jnp.float32),
        pltpu.VMEM((CHUNK, D), jnp.float32),
        pltpu.VMEM((CHUNK, D), jnp.float32),
        pltpu.SemaphoreType.DMA,
        pltpu.SemaphoreType.DMA,
        pltpu.SemaphoreType.DMA,
        pltpu.SemaphoreType.DMA,
    ],
)
def _gather(x_hbm, src_hbm, dst_hbm, xi_hbm, xj_hbm, idxs_s, idxs_d,
            ri0, rj0, ri1, rj1, g_sem0, g_sem1, wb_sem0, wb_sem1):
    wid = lax.axis_index("s") * NC + lax.axis_index("c")
    ebase = wid * EPW
    ri = (ri0, ri1)
    rj = (rj0, rj1)
    g_sem = (g_sem0, g_sem1)
    wb_sem = (wb_sem0, wb_sem1)

    pltpu.sync_copy(src_hbm.at[pl.ds(ebase, EPW)], idxs_s)
    pltpu.sync_copy(dst_hbm.at[pl.ds(ebase, EPW)], idxs_d)

    def fire_gathers(i, b, n):
        off = i * CHUNK
        pltpu.async_copy(x_hbm.at[idxs_d.at[pl.ds(off, n)]],
                         ri[b].at[pl.ds(0, n)], g_sem[b])
        pltpu.async_copy(x_hbm.at[idxs_s.at[pl.ds(off, n)]],
                         rj[b].at[pl.ds(0, n)], g_sem[b])

    def wait_gathers(i, b, n):
        off = i * CHUNK
        pltpu.make_async_copy(x_hbm.at[idxs_d.at[pl.ds(off, n)]],
                              ri[b].at[pl.ds(0, n)], g_sem[b]).wait()
        pltpu.make_async_copy(x_hbm.at[idxs_s.at[pl.ds(off, n)]],
                              rj[b].at[pl.ds(0, n)], g_sem[b]).wait()

    def fire_wb(i, b, n):
        base = ebase + i * CHUNK
        pltpu.async_copy(ri[b].at[pl.ds(0, n)], xi_hbm.at[pl.ds(base, n)],
                         wb_sem[b])
        pltpu.async_copy(rj[b].at[pl.ds(0, n)], xj_hbm.at[pl.ds(base, n)],
                         wb_sem[b])

    def wait_wb(b, n):
        pltpu.make_async_copy(ri[b].at[pl.ds(0, n)],
                              xi_hbm.at[pl.ds(ebase, n)], wb_sem[b]).wait()
        pltpu.make_async_copy(rj[b].at[pl.ds(0, n)],
                              xj_hbm.at[pl.ds(ebase, n)], wb_sem[b]).wait()

    fire_gathers(0, 0, CHUNK)

    def pair_body(p, _):
        for b in (0, 1):
            i = 2 * p + b
            pl.when(i >= 2)(lambda: wait_wb(b, CHUNK))
            pl.when(i >= 1)(lambda: fire_gathers(i, b, CHUNK))
            prev = i - 1
            pb = 1 - b

            def do_prev():
                wait_gathers(prev, pb, CHUNK)
                fire_wb(prev, pb, CHUNK)

            pl.when(i >= 1)(do_prev)
        return 0

    lax.fori_loop(0, NFULL // 2, pair_body, 0)

    wait_wb(0, CHUNK)
    fire_gathers(NFULL - 1, 0, CHUNK)
    wait_gathers(NFULL - 2, 1, CHUNK)
    fire_wb(NFULL - 2, 1, CHUNK)
    wait_gathers(NFULL - 1, 0, CHUNK)
    fire_wb(NFULL - 1, 0, CHUNK)
    wait_wb(1, CHUNK)
    fire_gathers(NFULL, 1, TAIL)
    wait_gathers(NFULL, 1, TAIL)
    fire_wb(NFULL, 1, TAIL)
    wait_wb(0, CHUNK)
    wait_wb(1, TAIL)


BE = 1280


def _mlp_body(xi_ref, xj_ref, ea_ref, w_ref, bf_ref, bs_ref, m_ref):
    xi = xi_ref[...]
    xj = xj_ref[...]
    ea = ea_ref[...]
    dot = lambda a, b: jax.lax.dot_general(
        a, b, (((1,), (0,)), ((), ())), preferred_element_type=jnp.float32)
    f = (dot(xi, w_ref[0]) + dot(xj, w_ref[1]) + dot(ea, w_ref[2])) + bf_ref[...]
    s = (dot(xi, w_ref[3]) + dot(xj, w_ref[4]) + dot(ea, w_ref[5])) + bs_ref[...]
    ef = jnp.exp(-jnp.abs(f))
    gate = jnp.where(f >= 0.0, 1.0 / (1.0 + ef), ef / (1.0 + ef))
    core = jnp.maximum(s, 0.0) + jnp.log1p(jnp.exp(-jnp.abs(s)))
    m_ref[...] = gate * core


def _edge_mlp(xi, xj, ea, w, bf2, bs2):
    return pl.pallas_call(
        _mlp_body,
        grid=(ESEG // BE,),
        in_specs=[
            pl.BlockSpec((BE, D), lambda i: (i, 0)),
            pl.BlockSpec((BE, D), lambda i: (i, 0)),
            pl.BlockSpec((BE, D), lambda i: (i, 0)),
            pl.BlockSpec((6, D, D), lambda i: (0, 0, 0)),
            pl.BlockSpec((1, D), lambda i: (0, 0)),
            pl.BlockSpec((1, D), lambda i: (0, 0)),
        ],
        out_specs=pl.BlockSpec((BE, D), lambda i: (i, 0)),
        out_shape=jax.ShapeDtypeStruct((ESEG, D), jnp.float32),
    )(xi, xj, ea, w, bf2, bs2)


@functools.partial(
    pl.kernel,
    out_type=jax.ShapeDtypeStruct((NC, N_PAD, D), jnp.float32),
    mesh=_MESH,
    scratch_types=[
        pltpu.VMEM((CHUNK,), jnp.int32),
        pltpu.VMEM((CHUNK,), jnp.int32),
        pltpu.VMEM((TAIL,), jnp.int32),
        pltpu.VMEM((CHUNK, D), jnp.float32),
        pltpu.VMEM((CHUNK, D), jnp.float32),
        pltpu.VMEM_SHARED((N_PAD, D), jnp.float32),
        pltpu.SemaphoreType.DMA,
        pltpu.SemaphoreType.DMA,
        pltpu.SemaphoreType.DMA,
        pltpu.SemaphoreType.DMA,
    ],
)
def _scatter(m_hbm, dst_hbm, z_hbm, p_hbm, idx0, idx1, idxt, m0, m1, acc_sh,
             ld_sem0, ld_sem1, add_sem0, add_sem1):
    cid = lax.axis_index("c")
    sid = lax.axis_index("s")
    wid = sid * NC + cid
    ebase = wid * EPW
    idxv = (idx0, idx1)
    mr = (m0, m1)
    ld_sem = (ld_sem0, ld_sem1)
    add_sem = (add_sem0, add_sem1)

    pltpu.sync_copy(z_hbm.at[pl.ds(sid * ROWS_PER_TILE, ROWS_PER_TILE)],
                    acc_sh.at[pl.ds(sid * ROWS_PER_TILE, ROWS_PER_TILE)])
    plsc.subcore_barrier()

    def fire_loads(i, b):
        base = ebase + i * CHUNK
        pltpu.async_copy(dst_hbm.at[pl.ds(base, CHUNK)], idxv[b], ld_sem[b])
        pltpu.async_copy(m_hbm.at[pl.ds(base, CHUNK)], mr[b], ld_sem[b])

    def wait_loads(i, b):
        base = ebase + i * CHUNK
        pltpu.make_async_copy(dst_hbm.at[pl.ds(base, CHUNK)], idxv[b],
                              ld_sem[b]).wait()
        pltpu.make_async_copy(m_hbm.at[pl.ds(base, CHUNK)], mr[b],
                              ld_sem[b]).wait()

    def fire_add(b):
        pltpu.async_copy(mr[b], acc_sh.at[idxv[b]], add_sem[b], add=True)

    def wait_add(b):
        pltpu.make_async_copy(mr[b], acc_sh.at[idxv[b]], add_sem[b]).wait()

    fire_loads(0, 0)

    def pair_body(p, _):
        for b in (0, 1):
            i = 2 * p + b
            pl.when(i >= 2)(lambda: wait_add(b))
            pl.when(i >= 1)(lambda: fire_loads(i, b))
            prev = i - 1
            pb = 1 - b

            def do_prev():
                wait_loads(prev, pb)
                fire_add(pb)

            pl.when(i >= 1)(do_prev)
        return 0

    lax.fori_loop(0, NFULL // 2, pair_body, 0)

    wait_add(0)
    fire_loads(NFULL - 1, 0)
    wait_loads(NFULL - 2, 1)
    fire_add(1)
    wait_loads(NFULL - 1, 0)
    fire_add(0)
    wait_add(1)
    tbase = ebase + NFULL * CHUNK
    pltpu.async_copy(dst_hbm.at[pl.ds(tbase, TAIL)], idxt, ld_sem1)
    pltpu.async_copy(m_hbm.at[pl.ds(tbase, TAIL)], m1.at[pl.ds(0, TAIL)], ld_sem1)
    pltpu.make_async_copy(dst_hbm.at[pl.ds(tbase, TAIL)], idxt, ld_sem1).wait()
    pltpu.make_async_copy(m_hbm.at[pl.ds(tbase, TAIL)], m1.at[pl.ds(0, TAIL)],
                          ld_sem1).wait()
    pltpu.async_copy(m1.at[pl.ds(0, TAIL)], acc_sh.at[idxt], add_sem1, add=True)
    wait_add(0)
    pltpu.make_async_copy(m1.at[pl.ds(0, TAIL)], acc_sh.at[idxt], add_sem1).wait()

    plsc.subcore_barrier()
    pltpu.sync_copy(acc_sh.at[pl.ds(sid * ROWS_PER_TILE, ROWS_PER_TILE)],
                    p_hbm.at[cid, pl.ds(sid * ROWS_PER_TILE, ROWS_PER_TILE)])


BN = 2000


def _combine_body(x_ref, *refs):
    p_refs = refs[:-1]
    o_ref = refs[-1]
    acc = x_ref[...]
    for p_ref in p_refs:
        acc = acc + p_ref[0] + p_ref[1]
    o_ref[...] = acc


def _combine(x, ps):
    return pl.pallas_call(
        _combine_body,
        grid=(N // BN,),
        in_specs=[pl.BlockSpec((BN, D), lambda i: (i, 0))] + [
            pl.BlockSpec((NC, BN, D), lambda i: (0, i, 0)) for _ in ps
        ],
        out_specs=pl.BlockSpec((BN, D), lambda i: (i, 0)),
        out_shape=jax.ShapeDtypeStruct((N, D), jnp.float32),
    )(x, *ps)


def kernel(x, edge_index, edge_attr, Wf, bf, Ws, bs):
    src = edge_index[0].astype(jnp.int32)
    dst = edge_index[1].astype(jnp.int32)

    w = jnp.stack([
        Wf[:, 0 * D:1 * D].T, Wf[:, 1 * D:2 * D].T, Wf[:, 2 * D:3 * D].T,
        Ws[:, 0 * D:1 * D].T, Ws[:, 1 * D:2 * D].T, Ws[:, 2 * D:3 * D].T,
    ])
    bf2 = bf.reshape(1, D)
    bs2 = bs.reshape(1, D)
    z = jnp.zeros((N_PAD, D), jnp.float32)

    ps = []
    for q in range(Q):
        sl = slice(q * ESEG, (q + 1) * ESEG)
        src_q, dst_q, ea_q = src[sl], dst[sl], edge_attr[sl]
        xi, xj = _gather(x, src_q, dst_q)
        m = _edge_mlp(xi, xj, ea_q, w, bf2, bs2)
        ps.append(_scatter(m, dst_q, z))
    return _combine(x, ps)


# speedup vs baseline: 1.1547x; 1.1547x over previous
import functools

import jax
import jax.numpy as jnp
from jax import lax
from jax.experimental import pallas as pl
from jax.experimental.pallas import tpu as pltpu
from jax.experimental.pallas import tpu_sc as plsc

N = 10000
E = 320000
D = 128

_info = plsc.get_sparse_core_info()
NC = _info.num_cores
NS = _info.num_subcores
NW = NC * NS

CHUNK = 128
NCH = E // CHUNK

_MESH = plsc.VectorSubcoreMesh(core_axis_name="c", subcore_axis_name="s")


EPW = E // NW
NFULL = EPW // CHUNK
TAIL = EPW - NFULL * CHUNK


@functools.partial(
    pl.kernel,
    out_type=(
        jax.ShapeDtypeStruct((E, D), jnp.float32),
        jax.ShapeDtypeStruct((E, D), jnp.float32),
    ),
    mesh=_MESH,
    scratch_types=[
        pltpu.VMEM((EPW,), jnp.int32),
        pltpu.VMEM((EPW,), jnp.int32),
        pltpu.VMEM((CHUNK, D), jnp.float32),
        pltpu.VMEM((CHUNK, D), jnp.float32),
        pltpu.VMEM((CHUNK, D), jnp.float32),
        pltpu.VMEM((CHUNK, D), jnp.float32),
        pltpu.SemaphoreType.DMA,
        pltpu.SemaphoreType.DMA,
        pltpu.SemaphoreType.DMA,
        pltpu.SemaphoreType.DMA,
    ],
)
def _gather(x_hbm, src_hbm, dst_hbm, xi_hbm, xj_hbm, idxs_s, idxs_d,
            ri0, rj0, ri1, rj1, g_sem0, g_sem1, wb_sem0, wb_sem1):
    wid = lax.axis_index("s") * NC + lax.axis_index("c")
    ebase = wid * EPW
    ri = (ri0, ri1)
    rj = (rj0, rj1)
    g_sem = (g_sem0, g_sem1)
    wb_sem = (wb_sem0, wb_sem1)

    pltpu.sync_copy(src_hbm.at[pl.ds(ebase, EPW)], idxs_s)
    pltpu.sync_copy(dst_hbm.at[pl.ds(ebase, EPW)], idxs_d)

    def fire_gathers(i, b, n):
        off = i * CHUNK
        pltpu.async_copy(x_hbm.at[idxs_d.at[pl.ds(off, n)]],
                         ri[b].at[pl.ds(0, n)], g_sem[b])
        pltpu.async_copy(x_hbm.at[idxs_s.at[pl.ds(off, n)]],
                         rj[b].at[pl.ds(0, n)], g_sem[b])

    def wait_gathers(i, b, n):
        off = i * CHUNK
        pltpu.make_async_copy(x_hbm.at[idxs_d.at[pl.ds(off, n)]],
                              ri[b].at[pl.ds(0, n)], g_sem[b]).wait()
        pltpu.make_async_copy(x_hbm.at[idxs_s.at[pl.ds(off, n)]],
                              rj[b].at[pl.ds(0, n)], g_sem[b]).wait()

    def fire_wb(i, b, n):
        base = ebase + i * CHUNK
        pltpu.async_copy(ri[b].at[pl.ds(0, n)], xi_hbm.at[pl.ds(base, n)],
                         wb_sem[b])
        pltpu.async_copy(rj[b].at[pl.ds(0, n)], xj_hbm.at[pl.ds(base, n)],
                         wb_sem[b])

    def wait_wb(b, n):
        pltpu.make_async_copy(ri[b].at[pl.ds(0, n)],
                              xi_hbm.at[pl.ds(ebase, n)], wb_sem[b]).wait()
        pltpu.make_async_copy(rj[b].at[pl.ds(0, n)],
                              xj_hbm.at[pl.ds(ebase, n)], wb_sem[b]).wait()

    fire_gathers(0, 0, CHUNK)

    def pair_body(p, _):
        for b in (0, 1):
            i = 2 * p + b
            pl.when(i >= 2)(lambda: wait_wb(b, CHUNK))
            pl.when(i >= 1)(lambda: fire_gathers(i, b, CHUNK))
            prev = i - 1
            pb = 1 - b

            def do_prev():
                wait_gathers(prev, pb, CHUNK)
                fire_wb(prev, pb, CHUNK)

            pl.when(i >= 1)(do_prev)
        return 0

    lax.fori_loop(0, NFULL // 2, pair_body, 0)

    wait_gathers(NFULL - 1, 1, CHUNK)
    fire_wb(NFULL - 1, 1, CHUNK)
    wait_wb(0, CHUNK)
    fire_gathers(NFULL, 0, TAIL)
    wait_gathers(NFULL, 0, TAIL)
    fire_wb(NFULL, 0, TAIL)
    wait_wb(1, CHUNK)
    wait_wb(0, TAIL)


BE = 2560


def _mlp_body(xi_ref, xj_ref, ea_ref, w_ref, bf_ref, bs_ref, m_ref):
    xi = xi_ref[...]
    xj = xj_ref[...]
    ea = ea_ref[...]
    dot = lambda a, b: jax.lax.dot_general(
        a, b, (((1,), (0,)), ((), ())), preferred_element_type=jnp.float32)
    f = (dot(xi, w_ref[0]) + dot(xj, w_ref[1]) + dot(ea, w_ref[2])) + bf_ref[...]
    s = (dot(xi, w_ref[3]) + dot(xj, w_ref[4]) + dot(ea, w_ref[5])) + bs_ref[...]
    ef = jnp.exp(-jnp.abs(f))
    gate = jnp.where(f >= 0.0, 1.0 / (1.0 + ef), ef / (1.0 + ef))
    core = jnp.maximum(s, 0.0) + jnp.log1p(jnp.exp(-jnp.abs(s)))
    m_ref[...] = gate * core


def _edge_mlp(xi, xj, ea, w, bf2, bs2):
    return pl.pallas_call(
        _mlp_body,
        grid=(E // BE,),
        in_specs=[
            pl.BlockSpec((BE, D), lambda i: (i, 0)),
            pl.BlockSpec((BE, D), lambda i: (i, 0)),
            pl.BlockSpec((BE, D), lambda i: (i, 0)),
            pl.BlockSpec((6, D, D), lambda i: (0, 0, 0)),
            pl.BlockSpec((1, D), lambda i: (0, 0)),
            pl.BlockSpec((1, D), lambda i: (0, 0)),
        ],
        out_specs=pl.BlockSpec((BE, D), lambda i: (i, 0)),
        out_shape=jax.ShapeDtypeStruct((E, D), jnp.float32),
    )(xi, xj, ea, w, bf2, bs2)


E_PER_CORE = E // NC
NCH_C = E_PER_CORE // CHUNK
N_PAD = 10240
ROWS_PER_TILE = N_PAD // NS


@functools.partial(
    pl.kernel,
    out_type=jax.ShapeDtypeStruct((NC, N_PAD, D), jnp.float32),
    mesh=_MESH,
    scratch_types=[
        pltpu.VMEM((CHUNK,), jnp.int32),
        pltpu.VMEM((CHUNK,), jnp.int32),
        pltpu.VMEM((TAIL,), jnp.int32),
        pltpu.VMEM((CHUNK, D), jnp.float32),
        pltpu.VMEM((CHUNK, D), jnp.float32),
        pltpu.VMEM_SHARED((N_PAD, D), jnp.float32),
        pltpu.SemaphoreType.DMA,
        pltpu.SemaphoreType.DMA,
        pltpu.SemaphoreType.DMA,
        pltpu.SemaphoreType.DMA,
    ],
)
def _scatter(m_hbm, dst_hbm, z_hbm, p_hbm, idx0, idx1, idxt, m0, m1, acc_sh,
             ld_sem0, ld_sem1, add_sem0, add_sem1):
    cid = lax.axis_index("c")
    sid = lax.axis_index("s")
    wid = sid * NC + cid
    ebase = wid * EPW
    idxv = (idx0, idx1)
    mr = (m0, m1)
    ld_sem = (ld_sem0, ld_sem1)
    add_sem = (add_sem0, add_sem1)

    pltpu.sync_copy(z_hbm.at[pl.ds(sid * ROWS_PER_TILE, ROWS_PER_TILE)],
                    acc_sh.at[pl.ds(sid * ROWS_PER_TILE, ROWS_PER_TILE)])
    plsc.subcore_barrier()

    def fire_loads(i, b):
        base = ebase + i * CHUNK
        pltpu.async_copy(dst_hbm.at[pl.ds(base, CHUNK)], idxv[b], ld_sem[b])
        pltpu.async_copy(m_hbm.at[pl.ds(base, CHUNK)], mr[b], ld_sem[b])

    def wait_loads(i, b):
        base = ebase + i * CHUNK
        pltpu.make_async_copy(dst_hbm.at[pl.ds(base, CHUNK)], idxv[b],
                              ld_sem[b]).wait()
        pltpu.make_async_copy(m_hbm.at[pl.ds(base, CHUNK)], mr[b],
                              ld_sem[b]).wait()

    def fire_add(b):
        pltpu.async_copy(mr[b], acc_sh.at[idxv[b]], add_sem[b], add=True)

    def wait_add(b):
        pltpu.make_async_copy(mr[b], acc_sh.at[idxv[b]], add_sem[b]).wait()

    fire_loads(0, 0)

    def pair_body(p, _):
        for b in (0, 1):
            i = 2 * p + b
            pl.when(i >= 2)(lambda: wait_add(b))
            pl.when(i >= 1)(lambda: fire_loads(i, b))
            prev = i - 1
            pb = 1 - b

            def do_prev():
                wait_loads(prev, pb)
                fire_add(pb)

            pl.when(i >= 1)(do_prev)
        return 0

    lax.fori_loop(0, NFULL // 2, pair_body, 0)

    wait_loads(NFULL - 1, 1)
    fire_add(1)
    wait_add(0)
    tbase = ebase + NFULL * CHUNK
    pltpu.async_copy(dst_hbm.at[pl.ds(tbase, TAIL)], idxt, ld_sem0)
    pltpu.async_copy(m_hbm.at[pl.ds(tbase, TAIL)], m0.at[pl.ds(0, TAIL)], ld_sem0)
    pltpu.make_async_copy(dst_hbm.at[pl.ds(tbase, TAIL)], idxt, ld_sem0).wait()
    pltpu.make_async_copy(m_hbm.at[pl.ds(tbase, TAIL)], m0.at[pl.ds(0, TAIL)],
                          ld_sem0).wait()
    pltpu.async_copy(m0.at[pl.ds(0, TAIL)], acc_sh.at[idxt], add_sem0, add=True)
    pltpu.make_async_copy(m0.at[pl.ds(0, TAIL)], acc_sh.at[idxt], add_sem0).wait()
    wait_add(1)

    plsc.subcore_barrier()
    pltpu.sync_copy(acc_sh.at[pl.ds(sid * ROWS_PER_TILE, ROWS_PER_TILE)],
                    p_hbm.at[cid, pl.ds(sid * ROWS_PER_TILE, ROWS_PER_TILE)])


BN = 2000


def _combine_body(x_ref, p_ref, o_ref):
    o_ref[...] = x_ref[...] + p_ref[0] + p_ref[1]


def _combine(x, p):
    return pl.pallas_call(
        _combine_body,
        grid=(N // BN,),
        in_specs=[
            pl.BlockSpec((BN, D), lambda i: (i, 0)),
            pl.BlockSpec((NC, BN, D), lambda i: (0, i, 0)),
        ],
        out_specs=pl.BlockSpec((BN, D), lambda i: (i, 0)),
        out_shape=jax.ShapeDtypeStruct((N, D), jnp.float32),
    )(x, p)


def kernel(x, edge_index, edge_attr, Wf, bf, Ws, bs):
    src = edge_index[0].astype(jnp.int32)
    dst = edge_index[1].astype(jnp.int32)

    w = jnp.stack([
        Wf[:, 0 * D:1 * D].T, Wf[:, 1 * D:2 * D].T, Wf[:, 2 * D:3 * D].T,
        Ws[:, 0 * D:1 * D].T, Ws[:, 1 * D:2 * D].T, Ws[:, 2 * D:3 * D].T,
    ])
    bf2 = bf.reshape(1, D)
    bs2 = bs.reshape(1, D)

    xi, xj = _gather(x, src, dst)
    m = _edge_mlp(xi, xj, edge_attr, w, bf2, bs2)
    z = jnp.zeros((N_PAD, D), jnp.float32)
    p = _scatter(m, dst, z)
    return _combine(x, p)


# speedup vs baseline: 1.2005x; 1.0396x over previous
import functools

import jax
import jax.numpy as jnp
from jax import lax
from jax.experimental import pallas as pl
from jax.experimental.pallas import tpu as pltpu
from jax.experimental.pallas import tpu_sc as plsc

N = 10000
E = 320000
D = 128

_info = plsc.get_sparse_core_info()
NC = _info.num_cores
NS = _info.num_subcores
NW = NC * NS

CHUNK = 128
NCH = E // CHUNK

_MESH = plsc.VectorSubcoreMesh(core_axis_name="c", subcore_axis_name="s")


EPW = E // NW
NFULL = EPW // CHUNK
TAIL = EPW - NFULL * CHUNK


@functools.partial(
    pl.kernel,
    out_type=(
        jax.ShapeDtypeStruct((E, D), jnp.float32),
        jax.ShapeDtypeStruct((E, D), jnp.float32),
    ),
    mesh=_MESH,
    scratch_types=[
        pltpu.VMEM((EPW,), jnp.int32),
        pltpu.VMEM((EPW,), jnp.int32),
        pltpu.VMEM((CHUNK, D), jnp.float32),
        pltpu.VMEM((CHUNK, D), jnp.float32),
        pltpu.VMEM((CHUNK, D), jnp.float32),
        pltpu.VMEM((CHUNK, D), jnp.float32),
        pltpu.SemaphoreType.DMA,
        pltpu.SemaphoreType.DMA,
        pltpu.SemaphoreType.DMA,
        pltpu.SemaphoreType.DMA,
    ],
)
def _gather(x_hbm, src_hbm, dst_hbm, xi_hbm, xj_hbm, idxs_s, idxs_d,
            ri0, rj0, ri1, rj1, g_sem0, g_sem1, wb_sem0, wb_sem1):
    wid = lax.axis_index("s") * NC + lax.axis_index("c")
    ebase = wid * EPW
    ri = (ri0, ri1)
    rj = (rj0, rj1)
    g_sem = (g_sem0, g_sem1)
    wb_sem = (wb_sem0, wb_sem1)

    pltpu.sync_copy(src_hbm.at[pl.ds(ebase, EPW)], idxs_s)
    pltpu.sync_copy(dst_hbm.at[pl.ds(ebase, EPW)], idxs_d)

    def fire_gathers(i, b, n):
        off = i * CHUNK
        pltpu.async_copy(x_hbm.at[idxs_d.at[pl.ds(off, n)]],
                         ri[b].at[pl.ds(0, n)], g_sem[b])
        pltpu.async_copy(x_hbm.at[idxs_s.at[pl.ds(off, n)]],
                         rj[b].at[pl.ds(0, n)], g_sem[b])

    def wait_gathers(i, b, n):
        off = i * CHUNK
        pltpu.make_async_copy(x_hbm.at[idxs_d.at[pl.ds(off, n)]],
                              ri[b].at[pl.ds(0, n)], g_sem[b]).wait()
        pltpu.make_async_copy(x_hbm.at[idxs_s.at[pl.ds(off, n)]],
                              rj[b].at[pl.ds(0, n)], g_sem[b]).wait()

    def fire_wb(i, b, n):
        base = ebase + i * CHUNK
        pltpu.async_copy(ri[b].at[pl.ds(0, n)], xi_hbm.at[pl.ds(base, n)],
                         wb_sem[b])
        pltpu.async_copy(rj[b].at[pl.ds(0, n)], xj_hbm.at[pl.ds(base, n)],
                         wb_sem[b])

    def wait_wb(b, n):
        pltpu.make_async_copy(ri[b].at[pl.ds(0, n)],
                              xi_hbm.at[pl.ds(ebase, n)], wb_sem[b]).wait()
        pltpu.make_async_copy(rj[b].at[pl.ds(0, n)],
                              xj_hbm.at[pl.ds(ebase, n)], wb_sem[b]).wait()

    fire_gathers(0, 0, CHUNK)

    def pair_body(p, _):
        for b in (0, 1):
            i = 2 * p + b
            pl.when(i >= 2)(lambda: wait_wb(b, CHUNK))
            pl.when(i >= 1)(lambda: fire_gathers(i, b, CHUNK))
            prev = i - 1
            pb = 1 - b

            def do_prev():
                wait_gathers(prev, pb, CHUNK)
                fire_wb(prev, pb, CHUNK)

            pl.when(i >= 1)(do_prev)
        return 0

    lax.fori_loop(0, NFULL // 2, pair_body, 0)

    wait_gathers(NFULL - 1, 1, CHUNK)
    fire_wb(NFULL - 1, 1, CHUNK)
    wait_wb(0, CHUNK)
    fire_gathers(NFULL, 0, TAIL)
    wait_gathers(NFULL, 0, TAIL)
    fire_wb(NFULL, 0, TAIL)
    wait_wb(1, CHUNK)
    wait_wb(0, TAIL)


BE = 4000


def _mlp_body(xi_ref, xj_ref, ea_ref, w_ref, bf_ref, bs_ref, m_ref):
    xi = xi_ref[...]
    xj = xj_ref[...]
    ea = ea_ref[...]
    dot = lambda a, b: jax.lax.dot_general(
        a, b, (((1,), (0,)), ((), ())), preferred_element_type=jnp.float32)
    f = (dot(xi, w_ref[0]) + dot(xj, w_ref[1]) + dot(ea, w_ref[2])) + bf_ref[...]
    s = (dot(xi, w_ref[3]) + dot(xj, w_ref[4]) + dot(ea, w_ref[5])) + bs_ref[...]
    ef = jnp.exp(-jnp.abs(f))
    gate = jnp.where(f >= 0.0, 1.0 / (1.0 + ef), ef / (1.0 + ef))
    core = jnp.maximum(s, 0.0) + jnp.log1p(jnp.exp(-jnp.abs(s)))
    m_ref[...] = gate * core


def _edge_mlp(xi, xj, ea, w, bf2, bs2):
    return pl.pallas_call(
        _mlp_body,
        grid=(E // BE,),
        in_specs=[
            pl.BlockSpec((BE, D), lambda i: (i, 0)),
            pl.BlockSpec((BE, D), lambda i: (i, 0)),
            pl.BlockSpec((BE, D), lambda i: (i, 0)),
            pl.BlockSpec((6, D, D), lambda i: (0, 0, 0)),
            pl.BlockSpec((1, D), lambda i: (0, 0)),
            pl.BlockSpec((1, D), lambda i: (0, 0)),
        ],
        out_specs=pl.BlockSpec((BE, D), lambda i: (i, 0)),
        out_shape=jax.ShapeDtypeStruct((E, D), jnp.float32),
    )(xi, xj, ea, w, bf2, bs2)


E_PER_CORE = E // NC
NCH_C = E_PER_CORE // CHUNK
N_PAD = 10240
ROWS_PER_TILE = N_PAD // NS


@functools.partial(
    pl.kernel,
    out_type=jax.ShapeDtypeStruct((NC, N_PAD, D), jnp.float32),
    mesh=_MESH,
    scratch_types=[
        pltpu.VMEM((CHUNK,), jnp.int32),
        pltpu.VMEM((CHUNK,), jnp.int32),
        pltpu.VMEM((TAIL,), jnp.int32),
        pltpu.VMEM((CHUNK, D), jnp.float32),
        pltpu.VMEM((CHUNK, D), jnp.float32),
        pltpu.VMEM_SHARED((N_PAD, D), jnp.float32),
        pltpu.SemaphoreType.DMA,
        pltpu.SemaphoreType.DMA,
        pltpu.SemaphoreType.DMA,
        pltpu.SemaphoreType.DMA,
    ],
)
def _scatter(m_hbm, dst_hbm, z_hbm, p_hbm, idx0, idx1, idxt, m0, m1, acc_sh,
             ld_sem0, ld_sem1, add_sem0, add_sem1):
    cid = lax.axis_index("c")
    sid = lax.axis_index("s")
    wid = sid * NC + cid
    ebase = wid * EPW
    idxv = (idx0, idx1)
    mr = (m0, m1)
    ld_sem = (ld_sem0, ld_sem1)
    add_sem = (add_sem0, add_sem1)

    pltpu.sync_copy(z_hbm.at[pl.ds(sid * ROWS_PER_TILE, ROWS_PER_TILE)],
                    acc_sh.at[pl.ds(sid * ROWS_PER_TILE, ROWS_PER_TILE)])
    plsc.subcore_barrier()

    def fire_loads(i, b):
        base = ebase + i * CHUNK
        pltpu.async_copy(dst_hbm.at[pl.ds(base, CHUNK)], idxv[b], ld_sem[b])
        pltpu.async_copy(m_hbm.at[pl.ds(base, CHUNK)], mr[b], ld_sem[b])

    def wait_loads(i, b):
        base = ebase + i * CHUNK
        pltpu.make_async_copy(dst_hbm.at[pl.ds(base, CHUNK)], idxv[b],
                              ld_sem[b]).wait()
        pltpu.make_async_copy(m_hbm.at[pl.ds(base, CHUNK)], mr[b],
                              ld_sem[b]).wait()

    def fire_add(b):
        pltpu.async_copy(mr[b], acc_sh.at[idxv[b]], add_sem[b], add=True)

    def wait_add(b):
        pltpu.make_async_copy(mr[b], acc_sh.at[idxv[b]], add_sem[b]).wait()

    fire_loads(0, 0)

    def pair_body(p, _):
        for b in (0, 1):
            i = 2 * p + b
            pl.when(i >= 2)(lambda: wait_add(b))
            pl.when(i >= 1)(lambda: fire_loads(i, b))
            prev = i - 1
            pb = 1 - b

            def do_prev():
                wait_loads(prev, pb)
                fire_add(pb)

            pl.when(i >= 1)(do_prev)
        return 0

    lax.fori_loop(0, NFULL // 2, pair_body, 0)

    wait_loads(NFULL - 1, 1)
    fire_add(1)
    wait_add(0)
    tbase = ebase + NFULL * CHUNK
    pltpu.async_copy(dst_hbm.at[pl.ds(tbase, TAIL)], idxt, ld_sem0)
    pltpu.async_copy(m_hbm.at[pl.ds(tbase, TAIL)], m0.at[pl.ds(0, TAIL)], ld_sem0)
    pltpu.make_async_copy(dst_hbm.at[pl.ds(tbase, TAIL)], idxt, ld_sem0).wait()
    pltpu.make_async_copy(m_hbm.at[pl.ds(tbase, TAIL)], m0.at[pl.ds(0, TAIL)],
                          ld_sem0).wait()
    pltpu.async_copy(m0.at[pl.ds(0, TAIL)], acc_sh.at[idxt], add_sem0, add=True)
    pltpu.make_async_copy(m0.at[pl.ds(0, TAIL)], acc_sh.at[idxt], add_sem0).wait()
    wait_add(1)

    plsc.subcore_barrier()
    pltpu.sync_copy(acc_sh.at[pl.ds(sid * ROWS_PER_TILE, ROWS_PER_TILE)],
                    p_hbm.at[cid, pl.ds(sid * ROWS_PER_TILE, ROWS_PER_TILE)])


BN = 2000


def _combine_body(x_ref, p_ref, o_ref):
    o_ref[...] = x_ref[...] + p_ref[0] + p_ref[1]


def _combine(x, p):
    return pl.pallas_call(
        _combine_body,
        grid=(N // BN,),
        in_specs=[
            pl.BlockSpec((BN, D), lambda i: (i, 0)),
            pl.BlockSpec((NC, BN, D), lambda i: (0, i, 0)),
        ],
        out_specs=pl.BlockSpec((BN, D), lambda i: (i, 0)),
        out_shape=jax.ShapeDtypeStruct((N, D), jnp.float32),
    )(x, p)


def kernel(x, edge_index, edge_attr, Wf, bf, Ws, bs):
    src = edge_index[0].astype(jnp.int32)
    dst = edge_index[1].astype(jnp.int32)

    w = jnp.stack([
        Wf[:, 0 * D:1 * D].T, Wf[:, 1 * D:2 * D].T, Wf[:, 2 * D:3 * D].T,
        Ws[:, 0 * D:1 * D].T, Ws[:, 1 * D:2 * D].T, Ws[:, 2 * D:3 * D].T,
    ])
    bf2 = bf.reshape(1, D)
    bs2 = bs.reshape(1, D)

    xi, xj = _gather(x, src, dst)
    m = _edge_mlp(xi, xj, edge_attr, w, bf2, bs2)
    z = jnp.zeros((N_PAD, D), jnp.float32)
    p = _scatter(m, dst, z)
    return _combine(x, p)


# speedup vs baseline: 1.2438x; 1.0361x over previous
import functools

import jax
import jax.numpy as jnp
from jax import lax
from jax.experimental import pallas as pl
from jax.experimental.pallas import tpu as pltpu
from jax.experimental.pallas import tpu_sc as plsc

N = 10000
E = 320000
D = 128

_info = plsc.get_sparse_core_info()
NC = _info.num_cores
NS = _info.num_subcores
NW = NC * NS

CHUNK = 128
NCH = E // CHUNK

_MESH = plsc.VectorSubcoreMesh(core_axis_name="c", subcore_axis_name="s")


EPW = E // NW
NFULL = EPW // CHUNK
TAIL = EPW - NFULL * CHUNK


@functools.partial(
    pl.kernel,
    out_type=(
        jax.ShapeDtypeStruct((E, D), jnp.float32),
        jax.ShapeDtypeStruct((E, D), jnp.float32),
    ),
    mesh=_MESH,
    scratch_types=[
        pltpu.VMEM((EPW,), jnp.int32),
        pltpu.VMEM((EPW,), jnp.int32),
        pltpu.VMEM((CHUNK, D), jnp.float32),
        pltpu.VMEM((CHUNK, D), jnp.float32),
        pltpu.VMEM((CHUNK, D), jnp.float32),
        pltpu.VMEM((CHUNK, D), jnp.float32),
        pltpu.SemaphoreType.DMA,
        pltpu.SemaphoreType.DMA,
        pltpu.SemaphoreType.DMA,
        pltpu.SemaphoreType.DMA,
    ],
)
def _gather(x_hbm, src_hbm, dst_hbm, xi_hbm, xj_hbm, idxs_s, idxs_d,
            ri0, rj0, ri1, rj1, g_sem0, g_sem1, wb_sem0, wb_sem1):
    wid = lax.axis_index("s") * NC + lax.axis_index("c")
    ebase = wid * EPW
    ri = (ri0, ri1)
    rj = (rj0, rj1)
    g_sem = (g_sem0, g_sem1)
    wb_sem = (wb_sem0, wb_sem1)

    pltpu.sync_copy(src_hbm.at[pl.ds(ebase, EPW)], idxs_s)
    pltpu.sync_copy(dst_hbm.at[pl.ds(ebase, EPW)], idxs_d)

    def fire_gathers(i, b, n):
        off = i * CHUNK
        pltpu.async_copy(x_hbm.at[idxs_d.at[pl.ds(off, n)]],
                         ri[b].at[pl.ds(0, n)], g_sem[b])
        pltpu.async_copy(x_hbm.at[idxs_s.at[pl.ds(off, n)]],
                         rj[b].at[pl.ds(0, n)], g_sem[b])

    def wait_gathers(i, b, n):
        off = i * CHUNK
        pltpu.make_async_copy(x_hbm.at[idxs_d.at[pl.ds(off, n)]],
                              ri[b].at[pl.ds(0, n)], g_sem[b]).wait()
        pltpu.make_async_copy(x_hbm.at[idxs_s.at[pl.ds(off, n)]],
                              rj[b].at[pl.ds(0, n)], g_sem[b]).wait()

    def fire_wb(i, b, n):
        base = ebase + i * CHUNK
        pltpu.async_copy(ri[b].at[pl.ds(0, n)], xi_hbm.at[pl.ds(base, n)],
                         wb_sem[b])
        pltpu.async_copy(rj[b].at[pl.ds(0, n)], xj_hbm.at[pl.ds(base, n)],
                         wb_sem[b])

    def wait_wb(b, n):
        pltpu.make_async_copy(ri[b].at[pl.ds(0, n)],
                              xi_hbm.at[pl.ds(ebase, n)], wb_sem[b]).wait()
        pltpu.make_async_copy(rj[b].at[pl.ds(0, n)],
                              xj_hbm.at[pl.ds(ebase, n)], wb_sem[b]).wait()

    fire_gathers(0, 0, CHUNK)

    def pair_body(p, _):
        for b in (0, 1):
            i = 2 * p + b
            pl.when(i >= 2)(lambda: wait_wb(b, CHUNK))
            pl.when(i >= 1)(lambda: fire_gathers(i, b, CHUNK))
            prev = i - 1
            pb = 1 - b

            def do_prev():
                wait_gathers(prev, pb, CHUNK)
                fire_wb(prev, pb, CHUNK)

            pl.when(i >= 1)(do_prev)
        return 0

    lax.fori_loop(0, NFULL // 2, pair_body, 0)

    wait_gathers(NFULL - 1, 1, CHUNK)
    fire_wb(NFULL - 1, 1, CHUNK)
    wait_wb(0, CHUNK)
    fire_gathers(NFULL, 0, TAIL)
    wait_gathers(NFULL, 0, TAIL)
    fire_wb(NFULL, 0, TAIL)
    wait_wb(1, CHUNK)
    wait_wb(0, TAIL)


BE = 8000


def _mlp_body(xi_ref, xj_ref, ea_ref, w_ref, bf_ref, bs_ref, m_ref):
    xi = xi_ref[...]
    xj = xj_ref[...]
    ea = ea_ref[...]
    dot = lambda a, b: jax.lax.dot_general(
        a, b, (((1,), (0,)), ((), ())), preferred_element_type=jnp.float32)
    f = (dot(xi, w_ref[0]) + dot(xj, w_ref[1]) + dot(ea, w_ref[2])) + bf_ref[...]
    s = (dot(xi, w_ref[3]) + dot(xj, w_ref[4]) + dot(ea, w_ref[5])) + bs_ref[...]
    ef = jnp.exp(-jnp.abs(f))
    gate = jnp.where(f >= 0.0, 1.0 / (1.0 + ef), ef / (1.0 + ef))
    core = jnp.maximum(s, 0.0) + jnp.log1p(jnp.exp(-jnp.abs(s)))
    m_ref[...] = gate * core


def _edge_mlp(xi, xj, ea, w, bf2, bs2):
    return pl.pallas_call(
        _mlp_body,
        grid=(E // BE,),
        in_specs=[
            pl.BlockSpec((BE, D), lambda i: (i, 0)),
            pl.BlockSpec((BE, D), lambda i: (i, 0)),
            pl.BlockSpec((BE, D), lambda i: (i, 0)),
            pl.BlockSpec((6, D, D), lambda i: (0, 0, 0)),
            pl.BlockSpec((1, D), lambda i: (0, 0)),
            pl.BlockSpec((1, D), lambda i: (0, 0)),
        ],
        out_specs=pl.BlockSpec((BE, D), lambda i: (i, 0)),
        out_shape=jax.ShapeDtypeStruct((E, D), jnp.float32),
    )(xi, xj, ea, w, bf2, bs2)


E_PER_CORE = E // NC
NCH_C = E_PER_CORE // CHUNK
N_PAD = 10240
ROWS_PER_TILE = N_PAD // NS


@functools.partial(
    pl.kernel,
    out_type=jax.ShapeDtypeStruct((NC, N_PAD, D), jnp.float32),
    mesh=_MESH,
    scratch_types=[
        pltpu.VMEM((CHUNK,), jnp.int32),
        pltpu.VMEM((CHUNK,), jnp.int32),
        pltpu.VMEM((TAIL,), jnp.int32),
        pltpu.VMEM((CHUNK, D), jnp.float32),
        pltpu.VMEM((CHUNK, D), jnp.float32),
        pltpu.VMEM_SHARED((N_PAD, D), jnp.float32),
        pltpu.SemaphoreType.DMA,
        pltpu.SemaphoreType.DMA,
        pltpu.SemaphoreType.DMA,
        pltpu.SemaphoreType.DMA,
    ],
)
def _scatter(m_hbm, dst_hbm, z_hbm, p_hbm, idx0, idx1, idxt, m0, m1, acc_sh,
             ld_sem0, ld_sem1, add_sem0, add_sem1):
    cid = lax.axis_index("c")
    sid = lax.axis_index("s")
    wid = sid * NC + cid
    ebase = wid * EPW
    idxv = (idx0, idx1)
    mr = (m0, m1)
    ld_sem = (ld_sem0, ld_sem1)
    add_sem = (add_sem0, add_sem1)

    pltpu.sync_copy(z_hbm.at[pl.ds(sid * ROWS_PER_TILE, ROWS_PER_TILE)],
                    acc_sh.at[pl.ds(sid * ROWS_PER_TILE, ROWS_PER_TILE)])
    plsc.subcore_barrier()

    def fire_loads(i, b):
        base = ebase + i * CHUNK
        pltpu.async_copy(dst_hbm.at[pl.ds(base, CHUNK)], idxv[b], ld_sem[b])
        pltpu.async_copy(m_hbm.at[pl.ds(base, CHUNK)], mr[b], ld_sem[b])

    def wait_loads(i, b):
        base = ebase + i * CHUNK
        pltpu.make_async_copy(dst_hbm.at[pl.ds(base, CHUNK)], idxv[b],
                              ld_sem[b]).wait()
        pltpu.make_async_copy(m_hbm.at[pl.ds(base, CHUNK)], mr[b],
                              ld_sem[b]).wait()

    def fire_add(b):
        pltpu.async_copy(mr[b], acc_sh.at[idxv[b]], add_sem[b], add=True)

    def wait_add(b):
        pltpu.make_async_copy(mr[b], acc_sh.at[idxv[b]], add_sem[b]).wait()

    fire_loads(0, 0)

    def pair_body(p, _):
        for b in (0, 1):
            i = 2 * p + b
            pl.when(i >= 2)(lambda: wait_add(b))
            pl.when(i >= 1)(lambda: fire_loads(i, b))
            prev = i - 1
            pb = 1 - b

            def do_prev():
                wait_loads(prev, pb)
                fire_add(pb)

            pl.when(i >= 1)(do_prev)
        return 0

    lax.fori_loop(0, NFULL // 2, pair_body, 0)

    wait_loads(NFULL - 1, 1)
    fire_add(1)
    wait_add(0)
    tbase = ebase + NFULL * CHUNK
    pltpu.async_copy(dst_hbm.at[pl.ds(tbase, TAIL)], idxt, ld_sem0)
    pltpu.async_copy(m_hbm.at[pl.ds(tbase, TAIL)], m0.at[pl.ds(0, TAIL)], ld_sem0)
    pltpu.make_async_copy(dst_hbm.at[pl.ds(tbase, TAIL)], idxt, ld_sem0).wait()
    pltpu.make_async_copy(m_hbm.at[pl.ds(tbase, TAIL)], m0.at[pl.ds(0, TAIL)],
                          ld_sem0).wait()
    pltpu.async_copy(m0.at[pl.ds(0, TAIL)], acc_sh.at[idxt], add_sem0, add=True)
    pltpu.make_async_copy(m0.at[pl.ds(0, TAIL)], acc_sh.at[idxt], add_sem0).wait()
    wait_add(1)

    plsc.subcore_barrier()
    pltpu.sync_copy(acc_sh.at[pl.ds(sid * ROWS_PER_TILE, ROWS_PER_TILE)],
                    p_hbm.at[cid, pl.ds(sid * ROWS_PER_TILE, ROWS_PER_TILE)])


BN = 2000


def _combine_body(x_ref, p_ref, o_ref):
    o_ref[...] = x_ref[...] + p_ref[0] + p_ref[1]


def _combine(x, p):
    return pl.pallas_call(
        _combine_body,
        grid=(N // BN,),
        in_specs=[
            pl.BlockSpec((BN, D), lambda i: (i, 0)),
            pl.BlockSpec((NC, BN, D), lambda i: (0, i, 0)),
        ],
        out_specs=pl.BlockSpec((BN, D), lambda i: (i, 0)),
        out_shape=jax.ShapeDtypeStruct((N, D), jnp.float32),
    )(x, p)


def kernel(x, edge_index, edge_attr, Wf, bf, Ws, bs):
    src = edge_index[0].astype(jnp.int32)
    dst = edge_index[1].astype(jnp.int32)

    w = jnp.stack([
        Wf[:, 0 * D:1 * D].T, Wf[:, 1 * D:2 * D].T, Wf[:, 2 * D:3 * D].T,
        Ws[:, 0 * D:1 * D].T, Ws[:, 1 * D:2 * D].T, Ws[:, 2 * D:3 * D].T,
    ])
    bf2 = bf.reshape(1, D)
    bs2 = bs.reshape(1, D)

    xi, xj = _gather(x, src, dst)
    m = _edge_mlp(xi, xj, edge_attr, w, bf2, bs2)
    z = jnp.zeros((N_PAD, D), jnp.float32)
    p = _scatter(m, dst, z)
    return _combine(x, p)


# speedup vs baseline: 1.2454x; 1.0014x over previous
import functools

import jax
import jax.numpy as jnp
from jax import lax
from jax.experimental import pallas as pl
from jax.experimental.pallas import tpu as pltpu
from jax.experimental.pallas import tpu_sc as plsc

N = 10000
E = 320000
D = 128

_info = plsc.get_sparse_core_info()
NC = _info.num_cores
NS = _info.num_subcores
NW = NC * NS

CHUNK = 128
NCH = E // CHUNK

_MESH = plsc.VectorSubcoreMesh(core_axis_name="c", subcore_axis_name="s")


EPW = E // NW
NFULL = EPW // CHUNK
TAIL = EPW - NFULL * CHUNK


@functools.partial(
    pl.kernel,
    out_type=(
        jax.ShapeDtypeStruct((E, D), jnp.float32),
        jax.ShapeDtypeStruct((E, D), jnp.float32),
    ),
    mesh=_MESH,
    scratch_types=[
        pltpu.VMEM((EPW,), jnp.int32),
        pltpu.VMEM((EPW,), jnp.int32),
        pltpu.VMEM((CHUNK, D), jnp.float32),
        pltpu.VMEM((CHUNK, D), jnp.float32),
        pltpu.VMEM((CHUNK, D), jnp.float32),
        pltpu.VMEM((CHUNK, D), jnp.float32),
        pltpu.SemaphoreType.DMA,
        pltpu.SemaphoreType.DMA,
        pltpu.SemaphoreType.DMA,
        pltpu.SemaphoreType.DMA,
    ],
)
def _gather(x_hbm, src_hbm, dst_hbm, xi_hbm, xj_hbm, idxs_s, idxs_d,
            ri0, rj0, ri1, rj1, g_sem0, g_sem1, wb_sem0, wb_sem1):
    wid = lax.axis_index("s") * NC + lax.axis_index("c")
    ebase = wid * EPW
    ri = (ri0, ri1)
    rj = (rj0, rj1)
    g_sem = (g_sem0, g_sem1)
    wb_sem = (wb_sem0, wb_sem1)

    pltpu.sync_copy(src_hbm.at[pl.ds(ebase, EPW)], idxs_s)
    pltpu.sync_copy(dst_hbm.at[pl.ds(ebase, EPW)], idxs_d)

    def fire_gathers(i, b, n):
        off = i * CHUNK
        pltpu.async_copy(x_hbm.at[idxs_d.at[pl.ds(off, n)]],
                         ri[b].at[pl.ds(0, n)], g_sem[b])
        pltpu.async_copy(x_hbm.at[idxs_s.at[pl.ds(off, n)]],
                         rj[b].at[pl.ds(0, n)], g_sem[b])

    def wait_gathers(i, b, n):
        off = i * CHUNK
        pltpu.make_async_copy(x_hbm.at[idxs_d.at[pl.ds(off, n)]],
                              ri[b].at[pl.ds(0, n)], g_sem[b]).wait()
        pltpu.make_async_copy(x_hbm.at[idxs_s.at[pl.ds(off, n)]],
                              rj[b].at[pl.ds(0, n)], g_sem[b]).wait()

    def fire_wb(i, b, n):
        base = ebase + i * CHUNK
        pltpu.async_copy(ri[b].at[pl.ds(0, n)], xi_hbm.at[pl.ds(base, n)],
                         wb_sem[b])
        pltpu.async_copy(rj[b].at[pl.ds(0, n)], xj_hbm.at[pl.ds(base, n)],
                         wb_sem[b])

    def wait_wb(b, n):
        pltpu.make_async_copy(ri[b].at[pl.ds(0, n)],
                              xi_hbm.at[pl.ds(ebase, n)], wb_sem[b]).wait()
        pltpu.make_async_copy(rj[b].at[pl.ds(0, n)],
                              xj_hbm.at[pl.ds(ebase, n)], wb_sem[b]).wait()

    fire_gathers(0, 0, CHUNK)

    def pair_body(p, _):
        for b in (0, 1):
            i = 2 * p + b
            pl.when(i >= 2)(lambda: wait_wb(b, CHUNK))
            pl.when(i >= 1)(lambda: fire_gathers(i, b, CHUNK))
            prev = i - 1
            pb = 1 - b

            def do_prev():
                wait_gathers(prev, pb, CHUNK)
                fire_wb(prev, pb, CHUNK)

            pl.when(i >= 1)(do_prev)
        return 0

    lax.fori_loop(0, NFULL // 2, pair_body, 0)

    wait_gathers(NFULL - 1, 1, CHUNK)
    fire_wb(NFULL - 1, 1, CHUNK)
    wait_wb(0, CHUNK)
    fire_gathers(NFULL, 0, TAIL)
    wait_gathers(NFULL, 0, TAIL)
    fire_wb(NFULL, 0, TAIL)
    wait_wb(1, CHUNK)
    wait_wb(0, TAIL)


BE = 10000


def _mlp_body(xi_ref, xj_ref, ea_ref, w_ref, bf_ref, bs_ref, m_ref):
    xi = xi_ref[...]
    xj = xj_ref[...]
    ea = ea_ref[...]
    dot = lambda a, b: jax.lax.dot_general(
        a, b, (((1,), (0,)), ((), ())), preferred_element_type=jnp.float32)
    f = (dot(xi, w_ref[0]) + dot(xj, w_ref[1]) + dot(ea, w_ref[2])) + bf_ref[...]
    s = (dot(xi, w_ref[3]) + dot(xj, w_ref[4]) + dot(ea, w_ref[5])) + bs_ref[...]
    ef = jnp.exp(-jnp.abs(f))
    gate = jnp.where(f >= 0.0, 1.0 / (1.0 + ef), ef / (1.0 + ef))
    core = jnp.maximum(s, 0.0) + jnp.log1p(jnp.exp(-jnp.abs(s)))
    m_ref[...] = gate * core


def _edge_mlp(xi, xj, ea, w, bf2, bs2):
    return pl.pallas_call(
        _mlp_body,
        grid=(E // BE,),
        in_specs=[
            pl.BlockSpec((BE, D), lambda i: (i, 0)),
            pl.BlockSpec((BE, D), lambda i: (i, 0)),
            pl.BlockSpec((BE, D), lambda i: (i, 0)),
            pl.BlockSpec((6, D, D), lambda i: (0, 0, 0)),
            pl.BlockSpec((1, D), lambda i: (0, 0)),
            pl.BlockSpec((1, D), lambda i: (0, 0)),
        ],
        out_specs=pl.BlockSpec((BE, D), lambda i: (i, 0)),
        out_shape=jax.ShapeDtypeStruct((E, D), jnp.float32),
    )(xi, xj, ea, w, bf2, bs2)


E_PER_CORE = E // NC
NCH_C = E_PER_CORE // CHUNK
N_PAD = 10240
ROWS_PER_TILE = N_PAD // NS


@functools.partial(
    pl.kernel,
    out_type=jax.ShapeDtypeStruct((NC, N_PAD, D), jnp.float32),
    mesh=_MESH,
    scratch_types=[
        pltpu.VMEM((CHUNK,), jnp.int32),
        pltpu.VMEM((CHUNK,), jnp.int32),
        pltpu.VMEM((TAIL,), jnp.int32),
        pltpu.VMEM((CHUNK, D), jnp.float32),
        pltpu.VMEM((CHUNK, D), jnp.float32),
        pltpu.VMEM_SHARED((N_PAD, D), jnp.float32),
        pltpu.SemaphoreType.DMA,
        pltpu.SemaphoreType.DMA,
        pltpu.SemaphoreType.DMA,
        pltpu.SemaphoreType.DMA,
    ],
)
def _scatter(m_hbm, dst_hbm, z_hbm, p_hbm, idx0, idx1, idxt, m0, m1, acc_sh,
             ld_sem0, ld_sem1, add_sem0, add_sem1):
    cid = lax.axis_index("c")
    sid = lax.axis_index("s")
    wid = sid * NC + cid
    ebase = wid * EPW
    idxv = (idx0, idx1)
    mr = (m0, m1)
    ld_sem = (ld_sem0, ld_sem1)
    add_sem = (add_sem0, add_sem1)

    pltpu.sync_copy(z_hbm.at[pl.ds(sid * ROWS_PER_TILE, ROWS_PER_TILE)],
                    acc_sh.at[pl.ds(sid * ROWS_PER_TILE, ROWS_PER_TILE)])
    plsc.subcore_barrier()

    def fire_loads(i, b):
        base = ebase + i * CHUNK
        pltpu.async_copy(dst_hbm.at[pl.ds(base, CHUNK)], idxv[b], ld_sem[b])
        pltpu.async_copy(m_hbm.at[pl.ds(base, CHUNK)], mr[b], ld_sem[b])

    def wait_loads(i, b):
        base = ebase + i * CHUNK
        pltpu.make_async_copy(dst_hbm.at[pl.ds(base, CHUNK)], idxv[b],
                              ld_sem[b]).wait()
        pltpu.make_async_copy(m_hbm.at[pl.ds(base, CHUNK)], mr[b],
                              ld_sem[b]).wait()

    def fire_add(b):
        pltpu.async_copy(mr[b], acc_sh.at[idxv[b]], add_sem[b], add=True)

    def wait_add(b):
        pltpu.make_async_copy(mr[b], acc_sh.at[idxv[b]], add_sem[b]).wait()

    fire_loads(0, 0)

    def pair_body(p, _):
        for b in (0, 1):
            i = 2 * p + b
            pl.when(i >= 2)(lambda: wait_add(b))
            pl.when(i >= 1)(lambda: fire_loads(i, b))
            prev = i - 1
            pb = 1 - b

            def do_prev():
                wait_loads(prev, pb)
                fire_add(pb)

            pl.when(i >= 1)(do_prev)
        return 0

    lax.fori_loop(0, NFULL // 2, pair_body, 0)

    wait_loads(NFULL - 1, 1)
    fire_add(1)
    wait_add(0)
    tbase = ebase + NFULL * CHUNK
    pltpu.async_copy(dst_hbm.at[pl.ds(tbase, TAIL)], idxt, ld_sem0)
    pltpu.async_copy(m_hbm.at[pl.ds(tbase, TAIL)], m0.at[pl.ds(0, TAIL)], ld_sem0)
    pltpu.make_async_copy(dst_hbm.at[pl.ds(tbase, TAIL)], idxt, ld_sem0).wait()
    pltpu.make_async_copy(m_hbm.at[pl.ds(tbase, TAIL)], m0.at[pl.ds(0, TAIL)],
                          ld_sem0).wait()
    pltpu.async_copy(m0.at[pl.ds(0, TAIL)], acc_sh.at[idxt], add_sem0, add=True)
    pltpu.make_async_copy(m0.at[pl.ds(0, TAIL)], acc_sh.at[idxt], add_sem0).wait()
    wait_add(1)

    plsc.subcore_barrier()
    pltpu.sync_copy(acc_sh.at[pl.ds(sid * ROWS_PER_TILE, ROWS_PER_TILE)],
                    p_hbm.at[cid, pl.ds(sid * ROWS_PER_TILE, ROWS_PER_TILE)])


BN = 2000


def _combine_body(x_ref, p_ref, o_ref):
    o_ref[...] = x_ref[...] + p_ref[0] + p_ref[1]


def _combine(x, p):
    return pl.pallas_call(
        _combine_body,
        grid=(N // BN,),
        in_specs=[
            pl.BlockSpec((BN, D), lambda i: (i, 0)),
            pl.BlockSpec((NC, BN, D), lambda i: (0, i, 0)),
        ],
        out_specs=pl.BlockSpec((BN, D), lambda i: (i, 0)),
        out_shape=jax.ShapeDtypeStruct((N, D), jnp.float32),
    )(x, p)


def kernel(x, edge_index, edge_attr, Wf, bf, Ws, bs):
    src = edge_index[0].astype(jnp.int32)
    dst = edge_index[1].astype(jnp.int32)

    w = jnp.stack([
        Wf[:, 0 * D:1 * D].T, Wf[:, 1 * D:2 * D].T, Wf[:, 2 * D:3 * D].T,
        Ws[:, 0 * D:1 * D].T, Ws[:, 1 * D:2 * D].T, Ws[:, 2 * D:3 * D].T,
    ])
    bf2 = bf.reshape(1, D)
    bs2 = bs.reshape(1, D)

    xi, xj = _gather(x, src, dst)
    m = _edge_mlp(xi, xj, edge_attr, w, bf2, bs2)
    z = jnp.zeros((N_PAD, D), jnp.float32)
    p = _scatter(m, dst, z)
    return _combine(x, p)
